# baseline (speedup 1.0000x reference)
import functools

import jax
import jax.numpy as jnp
from jax import lax
from jax.experimental import pallas as pl
from jax.experimental.pallas import tpu as pltpu
from jax.experimental.pallas import tpu_sc as plsc

N_BLK = 2048
PACK = 8


def _sc_gather(emb128, x_hi):
    B = x_hi.shape[0]
    D = emb128.shape[1]
    info = plsc.get_sparse_core_info()
    nw = info.num_cores * info.num_subcores
    b_per_w = B // nw

    mesh = plsc.VectorSubcoreMesh(core_axis_name="c", subcore_axis_name="s")

    @functools.partial(
        pl.kernel,
        mesh=mesh,
        out_type=jax.ShapeDtypeStruct((B, D), jnp.float32),
        scratch_types=[
            pltpu.VMEM((b_per_w,), jnp.int32),
            pltpu.VMEM((b_per_w, D), jnp.float32),
            pltpu.SemaphoreType.DMA,
        ],
    )
    def gather_kernel(emb_hbm, x_hbm, out_hbm, idx_v, rows_v, sem):
        wid = lax.axis_index("s") * info.num_cores + lax.axis_index("c")
        base = wid * b_per_w
        pltpu.sync_copy(x_hbm.at[pl.ds(base, b_per_w)], idx_v)
        pltpu.async_copy(emb_hbm.at[idx_v], rows_v, sem).wait()
        pltpu.sync_copy(rows_v, out_hbm.at[pl.ds(base, b_per_w)])

    return gather_kernel(emb128, x_hi)


def _mm_block(e128_ref, off_ref, wt_ref, b_ref, out_ref):
    off = off_ref[...]
    e = (off == 0.0) * e128_ref[:, 0:16]
    for k in range(1, PACK):
        e += (off == float(k)) * e128_ref[:, 16 * k : 16 * (k + 1)]
    out_ref[...] = (
        jnp.dot(e, wt_ref[...], preferred_element_type=jnp.float32) + b_ref[...]
    )


def _tc_matmul(e128, off, wt, b2d):
    B = e128.shape[0]
    D = wt.shape[0]
    N = wt.shape[1]
    grid = (pl.cdiv(N, N_BLK),)
    return pl.pallas_call(
        _mm_block,
        grid=grid,
        in_specs=[
            pl.BlockSpec((B, PACK * D), lambda j: (0, 0)),
            pl.BlockSpec((B, 1), lambda j: (0, 0)),
            pl.BlockSpec((D, N_BLK), lambda j: (0, j)),
            pl.BlockSpec((1, N_BLK), lambda j: (0, j)),
        ],
        out_specs=pl.BlockSpec((B, N_BLK), lambda j: (0, j)),
        out_shape=jax.ShapeDtypeStruct((B, N), jnp.float32),
    )(e128, off, wt, b2d)


def kernel(x, emb, fc_w, fc_b):
    V, D = emb.shape
    emb128 = emb.reshape(V // PACK, PACK * D)
    x_hi = (x >> 3).astype(jnp.int32)
    off = (x & 7).astype(jnp.float32).reshape(-1, 1)
    e128 = _sc_gather(emb128, x_hi)
    wt = fc_w.T
    return _tc_matmul(e128, off, wt, fc_b.reshape(1, -1))


# speedup vs baseline: 1.0320x; 1.0320x over previous
import functools

import jax
import jax.numpy as jnp
from jax import lax
from jax.experimental import pallas as pl
from jax.experimental.pallas import tpu as pltpu
from jax.experimental.pallas import tpu_sc as plsc

N_BLK = 2048
PACK = 8


def _sc_gather(emb128, x_hi):
    B = x_hi.shape[0]
    DP = emb128.shape[1]
    info = plsc.get_sparse_core_info()
    nw = info.num_cores * info.num_subcores
    b_per_w = B // nw

    mesh = plsc.VectorSubcoreMesh(core_axis_name="c", subcore_axis_name="s")

    @functools.partial(
        pl.kernel,
        mesh=mesh,
        out_type=jax.ShapeDtypeStruct((B, DP), jnp.float32),
        scratch_types=[
            pltpu.VMEM((b_per_w,), jnp.int32),
            pltpu.VMEM((b_per_w, DP), jnp.float32),
            pltpu.SemaphoreType.DMA,
        ],
    )
    def gather_kernel(emb_hbm, x_hbm, out_hbm, idx_v, rows_v, sem):
        wid = lax.axis_index("s") * info.num_cores + lax.axis_index("c")
        base = wid * b_per_w
        pltpu.sync_copy(x_hbm.at[pl.ds(base, b_per_w)], idx_v)
        pltpu.async_copy(emb_hbm.at[idx_v], rows_v, sem).wait()
        pltpu.sync_copy(rows_v, out_hbm.at[pl.ds(base, b_per_w)])

    return gather_kernel(emb128, x_hi)


def _mm_block(e128_ref, off_ref, wt_ref, b_ref, out_ref, e_ref):
    D = e_ref.shape[1]

    @pl.when(pl.program_id(0) == 0)
    def _select():
        off = off_ref[...]
        e = (off == 0.0) * e128_ref[:, 0:D]
        for k in range(1, PACK):
            e += (off == float(k)) * e128_ref[:, D * k : D * (k + 1)]
        e_ref[...] = e

    out_ref[...] = (
        jnp.dot(e_ref[...], wt_ref[...], preferred_element_type=jnp.float32)
        + b_ref[...]
    )


def _tc_matmul(e128, off, wt, b2d):
    B = e128.shape[0]
    D = wt.shape[0]
    N = wt.shape[1]
    grid = (pl.cdiv(N, N_BLK),)
    return pl.pallas_call(
        _mm_block,
        grid=grid,
        in_specs=[
            pl.BlockSpec((B, PACK * D), lambda j: (0, 0)),
            pl.BlockSpec((B, 1), lambda j: (0, 0)),
            pl.BlockSpec((D, N_BLK), lambda j: (0, j)),
            pl.BlockSpec((1, N_BLK), lambda j: (0, j)),
        ],
        out_specs=pl.BlockSpec((B, N_BLK), lambda j: (0, j)),
        out_shape=jax.ShapeDtypeStruct((B, N), jnp.float32),
        scratch_shapes=[pltpu.VMEM((B, D), jnp.float32)],
    )(e128, off, wt, b2d)


def kernel(x, emb, fc_w, fc_b):
    V, D = emb.shape
    emb128 = emb.reshape(V // PACK, PACK * D)
    x_hi = (x >> 3).astype(jnp.int32)
    off = (x & 7).astype(jnp.float32).reshape(-1, 1)
    e128 = _sc_gather(emb128, x_hi)
    wt = fc_w.T
    return _tc_matmul(e128, off, wt, fc_b.reshape(1, -1))
